# baseline (speedup 1.0000x reference)
import functools

import jax
import jax.numpy as jnp
from jax import lax
from jax.experimental import pallas as pl
from jax.experimental.pallas import tpu as pltpu
from jax.experimental.pallas import tpu_sc as plsc

ALPHA = 0.1
NITER = 10
NSC = 2
NTILE = 16
LANES = 16
CHUNK = 1024
PAD_ROWS = 16


def _mlp_body(x_ref, w1_ref, w2_ref, o_ref):
    h = lax.dot_general(
        x_ref[...], w1_ref[...], (((1,), (0,)), ((), ())),
        precision=lax.Precision.HIGHEST, preferred_element_type=jnp.float32)
    h = jnp.maximum(h, 0.0)
    o_ref[...] = lax.dot_general(
        h, w2_ref[...], (((1,), (0,)), ((), ())),
        precision=lax.Precision.HIGHEST, preferred_element_type=jnp.float32)


def _mlp(x, W1, W2):
    n, d_feat = x.shape
    hidden = W1.shape[1]
    nclasses = W2.shape[1]
    rows = 1000
    grid = n // rows
    return pl.pallas_call(
        _mlp_body,
        grid=(grid,),
        in_specs=[
            pl.BlockSpec((rows, d_feat), lambda i: (i, 0)),
            pl.BlockSpec((d_feat, hidden), lambda i: (0, 0)),
            pl.BlockSpec((hidden, nclasses), lambda i: (0, 0)),
        ],
        out_specs=pl.BlockSpec((rows, nclasses), lambda i: (i, 0)),
        out_shape=jax.ShapeDtypeStruct((n, nclasses), jnp.float32),
    )(x, W1, W2)


def _rsqrt_newton(d):
    i = plsc.bitcast(d, jnp.int32)
    i = 0x5F3759DF - lax.shift_right_logical(i, 1)
    y = plsc.bitcast(i, jnp.float32)
    for _ in range(3):
        y = y * (1.5 - 0.5 * d * y * y)
    return y


def _propagate(h3, ei_flat, n, e, nchunk):
    _, rows_pt, nclasses = h3.shape
    hc = nclasses // NSC
    tab_rows = n + PAD_ROWS
    out_pt = n // NTILE
    npair = rows_pt * hc // LANES
    ept = e // NTILE
    ept_pad = nchunk * CHUNK

    mesh = plsc.VectorSubcoreMesh(
        core_axis_name="c", subcore_axis_name="s",
        num_cores=NSC, num_subcores=NTILE)

    @functools.partial(
        pl.kernel,
        mesh=mesh,
        compiler_params=pltpu.CompilerParams(
            needs_layout_passes=False, use_tc_tiling_on_sc=False),
        out_type=jax.ShapeDtypeStruct((NTILE, out_pt, nclasses), jnp.float32),
        scratch_types=[
            pltpu.VMEM_SHARED((tab_rows, hc), jnp.float32),
            pltpu.VMEM_SHARED((tab_rows, hc), jnp.float32),
            pltpu.VMEM((nchunk * CHUNK,), jnp.int32),
            pltpu.VMEM((nchunk * CHUNK,), jnp.int32),
            pltpu.VMEM((CHUNK, hc), jnp.float32),
            pltpu.VMEM((CHUNK, hc), jnp.float32),
            pltpu.VMEM((CHUNK, hc), jnp.float32),
            pltpu.VMEM((rows_pt, hc), jnp.float32),
            pltpu.VMEM((rows_pt, nclasses), jnp.float32),
            pltpu.VMEM((rows_pt, hc), jnp.float32),
            pltpu.VMEM((rows_pt, hc), jnp.float32),
            pltpu.VMEM((npair, LANES), jnp.float32),
            pltpu.VMEM((npair, LANES), jnp.float32),
            pltpu.VMEM((npair, LANES), jnp.float32),
            pltpu.SemaphoreType.DMA,
            pltpu.SemaphoreType.DMA,
        ],
    )
    def k(h_r, ei_r, out_r,
          y_tab, s_tab, src_v, dst_v, rows_a, rows_b, ones_v, zeros_v,
          h_v, sb_v, yb_v, dx_v, c1_v, c2_v, sem_a, sem_b):
        wid = lax.axis_index("s")
        cid = lax.axis_index("c")
        r0 = wid * rows_pt
        lane = lax.iota(jnp.int32, LANES)
        col_ix = lax.bitwise_and(lane, jnp.int32(hc - 1))
        colh_ix = col_ix + hc * cid
        rowoff_ix = lax.shift_right_logical(lane, 3)

        def pair_load(ref, f, cols=col_ix):
            row_ix = rowoff_ix + 2 * f
            return plsc.load_gather(ref, [row_ix, cols])

        def pair_store(ref, f, val):
            row_ix = rowoff_ix + 2 * f
            plsc.store_scatter(ref, [row_ix, col_ix], val)

        pltpu.sync_copy(ei_r.at[pl.ds(wid * ept, ept)], src_v.at[pl.ds(0, ept)])
        pltpu.sync_copy(ei_r.at[pl.ds(e + wid * ept, ept)],
                        dst_v.at[pl.ds(0, ept)])
        pltpu.sync_copy(h_r.at[wid], h_v)

        def fill_pad(p, _):
            pos = ept + LANES * p + lane
            trash = jnp.int32(n) + lax.bitwise_and(lane, jnp.int32(PAD_ROWS - 1))
            plsc.store_scatter(src_v, [pos], trash)
            plsc.store_scatter(dst_v, [pos], trash)
            return 0
        lax.fori_loop(0, (ept_pad - ept) // LANES, fill_pad, 0)

        def fill_ones(i, _):
            pair_store(ones_v, i, jnp.full((LANES,), 1.0, jnp.float32))
            return 0
        lax.fori_loop(0, CHUNK * hc // LANES, fill_ones, 0)

        def fill_zeros(i, _):
            pair_store(zeros_v, i, jnp.zeros((LANES,), jnp.float32))
            return 0
        lax.fori_loop(0, npair, fill_zeros, 0)

        pltpu.sync_copy(zeros_v, s_tab.at[pl.ds(r0, rows_pt)])
        plsc.subcore_barrier()

        sems = (sem_a, sem_b)
        for j in range(2):
            pltpu.async_copy(
                ones_v, s_tab.at[dst_v.at[pl.ds(j * CHUNK, CHUNK)]],
                sems[j], add=True)

        def deg_pair(i, _):
            c = 2 * i
            for j in range(2):
                pltpu.make_async_copy(
                    ones_v,
                    s_tab.at[dst_v.at[pl.ds((c + j - 2) * CHUNK, CHUNK)]],
                    sems[j]).wait()
                pltpu.async_copy(
                    ones_v, s_tab.at[dst_v.at[pl.ds((c + j) * CHUNK, CHUNK)]],
                    sems[j], add=True)
            return 0
        lax.fori_loop(1, nchunk // 2, deg_pair, 0)
        for j in range(2):
            pltpu.make_async_copy(
                ones_v,
                s_tab.at[dst_v.at[pl.ds((nchunk - 2 + j) * CHUNK, CHUNK)]],
                sems[j]).wait()
        plsc.subcore_barrier()

        pltpu.sync_copy(s_tab.at[pl.ds(r0, rows_pt)], sb_v)
        pltpu.sync_copy(zeros_v, s_tab.at[pl.ds(r0, rows_pt)])

        def init_step(f, _):
            deg = pair_load(sb_v, f) + 1.0
            dx = _rsqrt_newton(deg)
            h16 = pair_load(h_v, f, colh_ix)
            dx_v[f, :] = dx
            c1_v[f, :] = (1.0 - ALPHA) * dx * dx
            c2_v[f, :] = ALPHA * dx * h16
            pair_store(sb_v, f, dx * h16)
            return 0
        lax.fori_loop(0, npair, init_step, 0)
        pltpu.sync_copy(sb_v, y_tab.at[pl.ds(r0, rows_pt)])
        plsc.subcore_barrier()

        def edge_phase():
            def sl(ref, c):
                return ref.at[pl.ds(c * CHUNK, CHUNK)]

            pltpu.async_copy(y_tab.at[sl(src_v, 0)], rows_a, sem_a)

            def edge_pair(i, _):
                c = 2 * i
                pltpu.async_copy(y_tab.at[sl(src_v, c + 1)], rows_b, sem_b)
                pltpu.make_async_copy(y_tab.at[sl(src_v, c)], rows_a, sem_a).wait()
                pltpu.sync_copy(rows_a, s_tab.at[sl(dst_v, c)], add=True)

                @pl.when(c + 2 < nchunk)
                def _():
                    pltpu.async_copy(y_tab.at[sl(src_v, c + 2)], rows_a, sem_a)

                pltpu.make_async_copy(y_tab.at[sl(src_v, c + 1)], rows_b, sem_b).wait()
                pltpu.sync_copy(rows_b, s_tab.at[sl(dst_v, c + 1)], add=True)
                return 0
            lax.fori_loop(0, nchunk // 2, edge_pair, 0)
            plsc.subcore_barrier()

        def update_prologue():
            pltpu.sync_copy(s_tab.at[pl.ds(r0, rows_pt)], sb_v)
            pltpu.sync_copy(y_tab.at[pl.ds(r0, rows_pt)], yb_v)
            pltpu.sync_copy(zeros_v, s_tab.at[pl.ds(r0, rows_pt)])

        for _ in range(NITER - 1):
            edge_phase()
            update_prologue()

            def upd_step(f, _):
                v = pair_load(sb_v, f) + pair_load(yb_v, f)
                pair_store(sb_v, f, c1_v[f, :] * v + c2_v[f, :])
                return 0
            lax.fori_loop(0, npair, upd_step, 0)
            pltpu.sync_copy(sb_v, y_tab.at[pl.ds(r0, rows_pt)])
            plsc.subcore_barrier()

        edge_phase()
        update_prologue()

        def final_step(f, _):
            v = pair_load(sb_v, f) + pair_load(yb_v, f)
            z = (c1_v[f, :] * v + c2_v[f, :]) / dx_v[f, :]
            pair_store(sb_v, f, z)
            return 0
        lax.fori_loop(0, npair, final_step, 0)
        pltpu.sync_copy(sb_v, y_tab.at[pl.ds(r0, rows_pt)])
        plsc.subcore_barrier()

        pltpu.sync_copy(y_tab.at[pl.ds(wid * out_pt, out_pt)],
                        sb_v.at[pl.ds(0, out_pt)])
        pltpu.sync_copy(sb_v.at[pl.ds(0, out_pt)],
                        out_r.at[wid, :, pl.ds(hc * cid, hc)])

    return k(h3, ei_flat)


def kernel(x, edge_index, W1, W2):
    n = x.shape[0]
    e = edge_index.shape[1]
    nclasses = W2.shape[1]
    assert n % NTILE == 0 and (n + PAD_ROWS) % NTILE == 0
    assert e % (NTILE * 8) == 0

    h = _mlp(x, W1, W2)

    rows_pt = (n + PAD_ROWS) // NTILE
    h3 = jnp.pad(h, ((0, PAD_ROWS), (0, 0))).reshape(NTILE, rows_pt, nclasses)
    ei_flat = edge_index.reshape(-1)

    per_tile = -(-e // (NTILE * 2 * CHUNK)) * (2 * CHUNK)
    out = _propagate(h3, ei_flat, n, e, per_tile // CHUNK)
    return out.reshape(n, nclasses)


# speedup vs baseline: 1.0936x; 1.0936x over previous
import functools

import jax
import jax.numpy as jnp
from jax import lax
from jax.experimental import pallas as pl
from jax.experimental.pallas import tpu as pltpu
from jax.experimental.pallas import tpu_sc as plsc

ALPHA = 0.1
NITER = 10
NSC = 2
NTILE = 16
LANES = 16
CHUNK = 512
PAD_ROWS = 16


def _mlp_body(x_ref, w1_ref, w2_ref, o_ref):
    h = lax.dot_general(
        x_ref[...], w1_ref[...], (((1,), (0,)), ((), ())),
        precision=lax.Precision.HIGHEST, preferred_element_type=jnp.float32)
    h = jnp.maximum(h, 0.0)
    o_ref[...] = lax.dot_general(
        h, w2_ref[...], (((1,), (0,)), ((), ())),
        precision=lax.Precision.HIGHEST, preferred_element_type=jnp.float32)


def _mlp(x, W1, W2):
    n, d_feat = x.shape
    hidden = W1.shape[1]
    nclasses = W2.shape[1]
    rows = 1000
    grid = n // rows
    return pl.pallas_call(
        _mlp_body,
        grid=(grid,),
        in_specs=[
            pl.BlockSpec((rows, d_feat), lambda i: (i, 0)),
            pl.BlockSpec((d_feat, hidden), lambda i: (0, 0)),
            pl.BlockSpec((hidden, nclasses), lambda i: (0, 0)),
        ],
        out_specs=pl.BlockSpec((rows, nclasses), lambda i: (i, 0)),
        out_shape=jax.ShapeDtypeStruct((n, nclasses), jnp.float32),
    )(x, W1, W2)


def _rsqrt_newton(d):
    i = plsc.bitcast(d, jnp.int32)
    i = 0x5F3759DF - lax.shift_right_logical(i, 1)
    y = plsc.bitcast(i, jnp.float32)
    for _ in range(3):
        y = y * (1.5 - 0.5 * d * y * y)
    return y


def _propagate(h3, ei_flat, n, e, nchunk):
    _, rows_pt, nclasses = h3.shape
    hc = nclasses // NSC
    tab_rows = n + PAD_ROWS
    out_pt = n // NTILE
    npair = rows_pt * hc // LANES
    ept = e // NTILE
    ept_pad = nchunk * CHUNK

    mesh = plsc.VectorSubcoreMesh(
        core_axis_name="c", subcore_axis_name="s",
        num_cores=NSC, num_subcores=NTILE)

    @functools.partial(
        pl.kernel,
        mesh=mesh,
        compiler_params=pltpu.CompilerParams(
            needs_layout_passes=False, use_tc_tiling_on_sc=False),
        out_type=jax.ShapeDtypeStruct((NTILE, out_pt, nclasses), jnp.float32),
        scratch_types=[
            pltpu.VMEM_SHARED((tab_rows, hc), jnp.float32),
            pltpu.VMEM_SHARED((tab_rows, hc), jnp.float32),
            pltpu.VMEM((nchunk * CHUNK,), jnp.int32),
            pltpu.VMEM((nchunk * CHUNK,), jnp.int32),
            pltpu.VMEM((CHUNK, hc), jnp.float32),
            pltpu.VMEM((CHUNK, hc), jnp.float32),
            pltpu.VMEM((CHUNK, hc), jnp.float32),
            pltpu.VMEM((rows_pt, hc), jnp.float32),
            pltpu.VMEM((rows_pt, nclasses), jnp.float32),
            pltpu.VMEM((rows_pt, hc), jnp.float32),
            pltpu.VMEM((rows_pt, hc), jnp.float32),
            pltpu.VMEM((npair, LANES), jnp.float32),
            pltpu.VMEM((npair, LANES), jnp.float32),
            pltpu.VMEM((npair, LANES), jnp.float32),
            pltpu.SemaphoreType.DMA,
            pltpu.SemaphoreType.DMA,
        ],
    )
    def k(h_r, ei_r, out_r,
          y_tab, s_tab, src_v, dst_v, rows_a, rows_b, ones_v, zeros_v,
          h_v, sb_v, yb_v, dx_v, c1_v, c2_v, sem_a, sem_b):
        wid = lax.axis_index("s")
        cid = lax.axis_index("c")
        r0 = wid * rows_pt
        lane = lax.iota(jnp.int32, LANES)
        col_ix = lax.bitwise_and(lane, jnp.int32(hc - 1))
        colh_ix = col_ix + hc * cid
        rowoff_ix = lax.shift_right_logical(lane, 3)

        def pair_load(ref, f, cols=col_ix):
            row_ix = rowoff_ix + 2 * f
            return plsc.load_gather(ref, [row_ix, cols])

        def pair_store(ref, f, val):
            row_ix = rowoff_ix + 2 * f
            plsc.store_scatter(ref, [row_ix, col_ix], val)

        pltpu.sync_copy(ei_r.at[pl.ds(wid * ept, ept)], src_v.at[pl.ds(0, ept)])
        pltpu.sync_copy(ei_r.at[pl.ds(e + wid * ept, ept)],
                        dst_v.at[pl.ds(0, ept)])
        pltpu.sync_copy(h_r.at[wid], h_v)

        def fill_pad(p, _):
            pos = ept + LANES * p + lane
            trash = jnp.int32(n) + lax.bitwise_and(lane, jnp.int32(PAD_ROWS - 1))
            plsc.store_scatter(src_v, [pos], trash)
            plsc.store_scatter(dst_v, [pos], trash)
            return 0
        lax.fori_loop(0, (ept_pad - ept) // LANES, fill_pad, 0)

        def fill_ones(i, _):
            pair_store(ones_v, i, jnp.full((LANES,), 1.0, jnp.float32))
            return 0
        lax.fori_loop(0, CHUNK * hc // LANES, fill_ones, 0)

        def fill_zeros(i, _):
            pair_store(zeros_v, i, jnp.zeros((LANES,), jnp.float32))
            return 0
        lax.fori_loop(0, npair, fill_zeros, 0)

        pltpu.sync_copy(zeros_v, s_tab.at[pl.ds(r0, rows_pt)])
        plsc.subcore_barrier()

        sems = (sem_a, sem_b)
        for j in range(2):
            pltpu.async_copy(
                ones_v, s_tab.at[dst_v.at[pl.ds(j * CHUNK, CHUNK)]],
                sems[j], add=True)

        def deg_pair(i, _):
            c = 2 * i
            for j in range(2):
                pltpu.make_async_copy(
                    ones_v,
                    s_tab.at[dst_v.at[pl.ds((c + j - 2) * CHUNK, CHUNK)]],
                    sems[j]).wait()
                pltpu.async_copy(
                    ones_v, s_tab.at[dst_v.at[pl.ds((c + j) * CHUNK, CHUNK)]],
                    sems[j], add=True)
            return 0
        lax.fori_loop(1, nchunk // 2, deg_pair, 0)
        for j in range(2):
            pltpu.make_async_copy(
                ones_v,
                s_tab.at[dst_v.at[pl.ds((nchunk - 2 + j) * CHUNK, CHUNK)]],
                sems[j]).wait()
        plsc.subcore_barrier()

        pltpu.sync_copy(s_tab.at[pl.ds(r0, rows_pt)], sb_v)
        pltpu.sync_copy(zeros_v, s_tab.at[pl.ds(r0, rows_pt)])

        @plsc.parallel_loop(0, npair, unroll=4)
        def _(f):
            deg = pair_load(sb_v, f) + 1.0
            dx = _rsqrt_newton(deg)
            h16 = pair_load(h_v, f, colh_ix)
            dx_v[f, :] = dx
            c1_v[f, :] = (1.0 - ALPHA) * dx * dx
            c2_v[f, :] = ALPHA * dx * h16
            pair_store(sb_v, f, dx * h16)
        pltpu.sync_copy(sb_v, y_tab.at[pl.ds(r0, rows_pt)])
        plsc.subcore_barrier()

        def edge_phase():
            def sl(ref, c):
                return ref.at[pl.ds(c * CHUNK, CHUNK)]

            pltpu.async_copy(y_tab.at[sl(src_v, 0)], rows_a, sem_a)

            def edge_pair(i, _):
                c = 2 * i
                pltpu.async_copy(y_tab.at[sl(src_v, c + 1)], rows_b, sem_b)
                pltpu.make_async_copy(y_tab.at[sl(src_v, c)], rows_a, sem_a).wait()
                pltpu.sync_copy(rows_a, s_tab.at[sl(dst_v, c)], add=True)

                @pl.when(c + 2 < nchunk)
                def _():
                    pltpu.async_copy(y_tab.at[sl(src_v, c + 2)], rows_a, sem_a)

                pltpu.make_async_copy(y_tab.at[sl(src_v, c + 1)], rows_b, sem_b).wait()
                pltpu.sync_copy(rows_b, s_tab.at[sl(dst_v, c + 1)], add=True)
                return 0
            lax.fori_loop(0, nchunk // 2, edge_pair, 0)
            plsc.subcore_barrier()

        def update_prologue():
            pltpu.sync_copy(s_tab.at[pl.ds(r0, rows_pt)], sb_v)
            pltpu.sync_copy(y_tab.at[pl.ds(r0, rows_pt)], yb_v)
            pltpu.sync_copy(zeros_v, s_tab.at[pl.ds(r0, rows_pt)])

        for _ in range(NITER - 1):
            edge_phase()
            update_prologue()

            @plsc.parallel_loop(0, npair, unroll=4)
            def _(f):
                v = pair_load(sb_v, f) + pair_load(yb_v, f)
                pair_store(sb_v, f, c1_v[f, :] * v + c2_v[f, :])
            pltpu.sync_copy(sb_v, y_tab.at[pl.ds(r0, rows_pt)])
            plsc.subcore_barrier()

        edge_phase()
        update_prologue()

        @plsc.parallel_loop(0, npair, unroll=4)
        def _(f):
            v = pair_load(sb_v, f) + pair_load(yb_v, f)
            z = (c1_v[f, :] * v + c2_v[f, :]) / dx_v[f, :]
            pair_store(sb_v, f, z)
        pltpu.sync_copy(sb_v, y_tab.at[pl.ds(r0, rows_pt)])
        plsc.subcore_barrier()

        pltpu.sync_copy(y_tab.at[pl.ds(wid * out_pt, out_pt)],
                        sb_v.at[pl.ds(0, out_pt)])
        pltpu.sync_copy(sb_v.at[pl.ds(0, out_pt)],
                        out_r.at[wid, :, pl.ds(hc * cid, hc)])

    return k(h3, ei_flat)


def kernel(x, edge_index, W1, W2):
    n = x.shape[0]
    e = edge_index.shape[1]
    nclasses = W2.shape[1]
    assert n % NTILE == 0 and (n + PAD_ROWS) % NTILE == 0
    assert e % (NTILE * 8) == 0

    h = _mlp(x, W1, W2)

    rows_pt = (n + PAD_ROWS) // NTILE
    h3 = jnp.pad(h, ((0, PAD_ROWS), (0, 0))).reshape(NTILE, rows_pt, nclasses)
    ei_flat = edge_index.reshape(-1)

    per_tile = -(-e // (NTILE * 2 * CHUNK)) * (2 * CHUNK)
    out = _propagate(h3, ei_flat, n, e, per_tile // CHUNK)
    return out.reshape(n, nclasses)
